# initial kernel scaffold (unmeasured)
import jax
import jax.numpy as jnp
from jax import lax
from jax.experimental import pallas as pl
from jax.experimental.pallas import tpu as pltpu


def kernel(
    x,
):
    def body(*refs):
        pass

    out_shape = jax.ShapeDtypeStruct(..., jnp.float32)
    return pl.pallas_call(body, out_shape=out_shape)(...)


# baseline (device time: 6562 ns/iter reference)
import jax
import jax.numpy as jnp
from jax import lax
from jax.experimental import pallas as pl
from jax.experimental.pallas import tpu as pltpu

N_DEV = 4


def kernel(x):
    m_per, n = x.shape

    def body(x_ref, out_ref, gather_ref, send_sems, recv_sems):
        my = lax.axis_index("i")

        barrier_sem = pltpu.get_barrier_semaphore()
        for off in (1, 2, 3):
            pl.semaphore_signal(
                barrier_sem, inc=1,
                device_id=((my + off) % N_DEV,),
                device_id_type=pl.DeviceIdType.MESH,
            )
        pl.semaphore_wait(barrier_sem, N_DEV - 1)

        xv = x_ref[:, :]
        vmax = jnp.max(xv, axis=0, keepdims=True)
        iota = lax.broadcasted_iota(jnp.int32, xv.shape, 0).astype(jnp.float32)
        lidx = jnp.min(jnp.where(xv == vmax, iota, 1e9), axis=0,
                       keepdims=True)
        gidx = lidx + my.astype(jnp.float32) * m_per
        chunk = jnp.concatenate([vmax, gidx], axis=0)
        gather_ref[pl.ds(my, 1), :, :] = chunk[None, :, :]

        rdmas = []
        for off in (1, 2, 3):
            rdma = pltpu.make_async_remote_copy(
                src_ref=gather_ref.at[my],
                dst_ref=gather_ref.at[my],
                send_sem=send_sems.at[off - 1],
                recv_sem=recv_sems.at[off - 1],
                device_id=((my + off) % N_DEV,),
                device_id_type=pl.DeviceIdType.MESH,
            )
            rdma.start()
            rdmas.append(rdma)
        for rdma in rdmas:
            rdma.wait()

        best_v = gather_ref[0, 0:1, :]
        best_i = gather_ref[0, 1:2, :]
        for s in range(1, N_DEV):
            v = gather_ref[s, 0:1, :]
            i = gather_ref[s, 1:2, :]
            take = v > best_v
            best_v = jnp.where(take, v, best_v)
            best_i = jnp.where(take, i, best_i)
        out_ref[0:1, :] = best_v
        out_ref[1:2, :] = best_i

    return pl.pallas_call(
        body,
        out_shape=jax.ShapeDtypeStruct((2, n), jnp.float32),
        in_specs=[pl.BlockSpec(memory_space=pltpu.VMEM)],
        out_specs=pl.BlockSpec(memory_space=pltpu.VMEM),
        scratch_shapes=[
            pltpu.VMEM((N_DEV, 2, n), jnp.float32),
            pltpu.SemaphoreType.DMA((N_DEV - 1,)),
            pltpu.SemaphoreType.DMA((N_DEV - 1,)),
        ],
        compiler_params=pltpu.CompilerParams(collective_id=0),
    )(x)


# device time: 6558 ns/iter; 1.0006x vs baseline; 1.0006x over previous
import jax
import jax.numpy as jnp
from jax import lax
from jax.experimental import pallas as pl
from jax.experimental.pallas import tpu as pltpu

N_DEV = 4


def kernel(x):
    m_per, n = x.shape

    def body(x_ref, out_ref, gather_ref, send_sems, recv_sems):
        my = lax.axis_index("i")

        xv = x_ref[:, :]
        vmax = jnp.max(xv, axis=0, keepdims=True)
        iota = lax.broadcasted_iota(jnp.int32, xv.shape, 0).astype(jnp.float32)
        lidx = jnp.min(jnp.where(xv == vmax, iota, 1e9), axis=0,
                       keepdims=True)
        gidx = lidx + my.astype(jnp.float32) * m_per
        chunk = jnp.concatenate([vmax, gidx], axis=0)
        gather_ref[pl.ds(my, 1), :, :] = chunk[None, :, :]

        barrier_sem = pltpu.get_barrier_semaphore()
        for off in (1, 2, 3):
            pl.semaphore_signal(
                barrier_sem, inc=1,
                device_id=((my + off) % N_DEV,),
                device_id_type=pl.DeviceIdType.MESH,
            )
        pl.semaphore_wait(barrier_sem, N_DEV - 1)

        rdmas = []
        for off in (1, 2, 3):
            rdma = pltpu.make_async_remote_copy(
                src_ref=gather_ref.at[my],
                dst_ref=gather_ref.at[my],
                send_sem=send_sems.at[off - 1],
                recv_sem=recv_sems.at[off - 1],
                device_id=((my + off) % N_DEV,),
                device_id_type=pl.DeviceIdType.MESH,
            )
            rdma.start()
            rdmas.append(rdma)
        for rdma in rdmas:
            rdma.wait_recv()

        best_v = gather_ref[0, 0:1, :]
        best_i = gather_ref[0, 1:2, :]
        for s in range(1, N_DEV):
            v = gather_ref[s, 0:1, :]
            i = gather_ref[s, 1:2, :]
            take = v > best_v
            best_v = jnp.where(take, v, best_v)
            best_i = jnp.where(take, i, best_i)
        out_ref[0:1, :] = best_v
        out_ref[1:2, :] = best_i

        for rdma in rdmas:
            rdma.wait_send()

    return pl.pallas_call(
        body,
        out_shape=jax.ShapeDtypeStruct((2, n), jnp.float32),
        in_specs=[pl.BlockSpec(memory_space=pltpu.VMEM)],
        out_specs=pl.BlockSpec(memory_space=pltpu.VMEM),
        scratch_shapes=[
            pltpu.VMEM((N_DEV, 2, n), jnp.float32),
            pltpu.SemaphoreType.DMA((N_DEV - 1,)),
            pltpu.SemaphoreType.DMA((N_DEV - 1,)),
        ],
        compiler_params=pltpu.CompilerParams(collective_id=0),
    )(x)


# device time: 6479 ns/iter; 1.0128x vs baseline; 1.0122x over previous
import jax
import jax.numpy as jnp
from jax import lax
from jax.experimental import pallas as pl
from jax.experimental.pallas import tpu as pltpu

N_DEV = 4


def kernel(x):
    m_per, n = x.shape

    def body(x_ref, out_ref, gather_ref, send_sems, recv_sems):
        my = lax.axis_index("i")

        barrier_sem = pltpu.get_barrier_semaphore()
        for off in (1, 2, 3):
            pl.semaphore_signal(
                barrier_sem, inc=1,
                device_id=((my + off) % N_DEV,),
                device_id_type=pl.DeviceIdType.MESH,
            )

        xv = x_ref[:, :]
        vmax = jnp.max(xv, axis=0, keepdims=True)
        iota = lax.broadcasted_iota(jnp.int32, xv.shape, 0).astype(jnp.float32)
        lidx = jnp.min(jnp.where(xv == vmax, iota, 1e9), axis=0,
                       keepdims=True)
        gidx = lidx + my.astype(jnp.float32) * m_per
        chunk = jnp.concatenate([vmax, gidx], axis=0)
        gather_ref[pl.ds(my, 1), :, :] = chunk[None, :, :]

        pl.semaphore_wait(barrier_sem, N_DEV - 1)

        rdmas = []
        for off in (2, 1, 3):
            rdma = pltpu.make_async_remote_copy(
                src_ref=gather_ref.at[my],
                dst_ref=gather_ref.at[my],
                send_sem=send_sems.at[off - 1],
                recv_sem=recv_sems.at[off - 1],
                device_id=((my + off) % N_DEV,),
                device_id_type=pl.DeviceIdType.MESH,
            )
            rdma.start()
            rdmas.append(rdma)
        for rdma in rdmas:
            rdma.wait_recv()

        best_v = gather_ref[0, 0:1, :]
        best_i = gather_ref[0, 1:2, :]
        for s in range(1, N_DEV):
            v = gather_ref[s, 0:1, :]
            i = gather_ref[s, 1:2, :]
            take = v > best_v
            best_v = jnp.where(take, v, best_v)
            best_i = jnp.where(take, i, best_i)
        out_ref[0:1, :] = best_v
        out_ref[1:2, :] = best_i

        for rdma in rdmas:
            rdma.wait_send()

    return pl.pallas_call(
        body,
        out_shape=jax.ShapeDtypeStruct((2, n), jnp.float32),
        in_specs=[pl.BlockSpec(memory_space=pltpu.VMEM)],
        out_specs=pl.BlockSpec(memory_space=pltpu.VMEM),
        scratch_shapes=[
            pltpu.VMEM((N_DEV, 2, n), jnp.float32),
            pltpu.SemaphoreType.DMA((N_DEV - 1,)),
            pltpu.SemaphoreType.DMA((N_DEV - 1,)),
        ],
        compiler_params=pltpu.CompilerParams(collective_id=0),
    )(x)
